# Initial kernel scaffold; baseline (speedup 1.0000x reference)
import jax
import jax.numpy as jnp
from jax.experimental import pallas as pl


def kernel(x, spiral, Wp, W1a, b1a, W1b, b1b, Wd, Wr0a, br0a, Wr0b, br0b, Wr1a, br1a, Wr1b, br1b, Wr2a, br2a, Wr2b, br2b, Wo1, bo1, Wo2, bo2, Wo3, bo3):
    raise NotImplementedError("write your pallas kernel here")


# speedup vs baseline: 1.0441x; 1.0441x over previous
import functools

import jax
import jax.numpy as jnp
from jax import lax
from jax.experimental import pallas as pl
from jax.experimental.pallas import tpu as pltpu
from jax.experimental.pallas import tpu_sc as plsc

V = 10000
VP = 10240
L = 16
NW = 32
GK = 128


def _make_sc_gather(C: int):
    R = VP * L
    chunk = R // NW
    iters = chunk // GK
    mesh = plsc.VectorSubcoreMesh(core_axis_name="c", subcore_axis_name="s")

    @functools.partial(
        pl.kernel,
        mesh=mesh,
        out_type=jax.ShapeDtypeStruct((R, C), jnp.float32),
        scratch_types=[
            pltpu.VMEM((chunk,), jnp.int32),
            pltpu.VMEM((GK, C), jnp.float32),
            pltpu.SemaphoreType.DMA,
        ],
    )
    def kfn(table_hbm, idx_hbm, g_hbm, idx_v, rows_v, sem):
        wid = lax.axis_index("s") * 2 + lax.axis_index("c")
        base = pl.multiple_of(wid * chunk, GK)
        pltpu.sync_copy(idx_hbm.at[pl.ds(base, chunk)], idx_v)

        def body(i, carry):
            off = pl.multiple_of(base + i * GK, GK)
            pltpu.async_copy(
                table_hbm.at[idx_v.at[pl.ds(i * GK, GK)]], rows_v, sem
            ).wait()
            pltpu.sync_copy(rows_v, g_hbm.at[pl.ds(off, GK)])
            return carry

        lax.fori_loop(0, iters, body, 0)

    return kfn


def _mm(xp, W, bias8=None, relu=False, res=None, mask=True, BM=256):
    M, K = xp.shape
    N = W.shape[1]
    nblk = M // BM
    args = [xp, W]
    in_specs = [
        pl.BlockSpec((BM, K), lambda i: (i, 0)),
        pl.BlockSpec((K, N), lambda i: (0, 0)),
    ]
    if bias8 is not None:
        args.append(bias8)
        in_specs.append(pl.BlockSpec((8, N), lambda i: (0, 0)))
    if res is not None:
        args.append(res)
        in_specs.append(pl.BlockSpec((BM, N), lambda i: (i, 0)))
    have_bias = bias8 is not None
    have_res = res is not None

    def body(*refs):
        x_ref, w_ref = refs[0], refs[1]
        rest = refs[2:-1]
        o_ref = refs[-1]
        y = jnp.dot(x_ref[...], w_ref[...], preferred_element_type=jnp.float32)
        ri = 0
        if have_bias:
            y = y + rest[0][0:1, :]
            ri = 1
        if mask:
            i = pl.program_id(0)
            rows = i * BM + lax.broadcasted_iota(jnp.int32, (BM, 1), 0)
            y = jnp.where(rows < V, y, 0.0)
        if have_res:
            y = y + rest[ri][...]
        if relu:
            y = jnp.maximum(y, 0.0)
        o_ref[...] = y

    return pl.pallas_call(
        body,
        grid=(nblk,),
        in_specs=in_specs,
        out_specs=pl.BlockSpec((BM, N), lambda i: (i, 0)),
        out_shape=jax.ShapeDtypeStruct((M, N), jnp.float32),
    )(*args)


def _wd_and_max(fs, Wd, BM=256):
    M, K = fs.shape
    N = Wd.shape[1]
    nblk = M // BM

    def body(x_ref, w_ref, o_ref, m_ref):
        i = pl.program_id(0)
        y = jnp.dot(x_ref[...], w_ref[...], preferred_element_type=jnp.float32)
        rows = i * BM + lax.broadcasted_iota(jnp.int32, (BM, 1), 0)
        valid = rows < V
        o_ref[...] = jnp.where(valid, y, 0.0)
        ym = jnp.where(valid, y, -1e30)
        m = ym[0:8]
        for j in range(1, BM // 8):
            m = jnp.maximum(m, ym[j * 8:(j + 1) * 8])

        @pl.when(i == 0)
        def _():
            m_ref[...] = m

        @pl.when(i > 0)
        def _():
            m_ref[...] = jnp.maximum(m_ref[...], m)

    return pl.pallas_call(
        body,
        grid=(nblk,),
        in_specs=[
            pl.BlockSpec((BM, K), lambda i: (i, 0)),
            pl.BlockSpec((K, N), lambda i: (0, 0)),
        ],
        out_specs=[
            pl.BlockSpec((BM, N), lambda i: (i, 0)),
            pl.BlockSpec((8, N), lambda i: (0, 0)),
        ],
        out_shape=[
            jax.ShapeDtypeStruct((M, N), jnp.float32),
            jax.ShapeDtypeStruct((8, N), jnp.float32),
        ],
    )(fs, Wd)


def _final_mlp(pfs, fs, macc, W1p, W1f, W1g, b1_8, W2, b2_8, W3p, b3_8, BM=256):
    M = pfs.shape[0]
    nblk = M // BM

    def body(p_ref, f_ref, g_ref, w1p, w1f, w1g, b1, w2, b2, w3, b3, o_ref):
        gmax = jnp.max(g_ref[...], axis=0, keepdims=True)
        gc = jnp.dot(gmax, w1g[...], preferred_element_type=jnp.float32)
        o1 = jnp.dot(p_ref[...], w1p[...], preferred_element_type=jnp.float32)
        o1 = o1 + jnp.dot(f_ref[...], w1f[...],
                          preferred_element_type=jnp.float32)
        o1 = jnp.maximum(o1 + gc + b1[0:1, :], 0.0)
        o2 = jnp.maximum(
            jnp.dot(o1, w2[...], preferred_element_type=jnp.float32)
            + b2[0:1, :], 0.0)
        o_ref[...] = (
            jnp.dot(o2, w3[...], preferred_element_type=jnp.float32)
            + b3[0:1, :])

    specs = [
        pl.BlockSpec((BM, 256), lambda i: (i, 0)),
        pl.BlockSpec((BM, 128), lambda i: (i, 0)),
        pl.BlockSpec((8, 128), lambda i: (0, 0)),
        pl.BlockSpec((256, 256), lambda i: (0, 0)),
        pl.BlockSpec((128, 256), lambda i: (0, 0)),
        pl.BlockSpec((128, 256), lambda i: (0, 0)),
        pl.BlockSpec((8, 256), lambda i: (0, 0)),
        pl.BlockSpec((256, 128), lambda i: (0, 0)),
        pl.BlockSpec((8, 128), lambda i: (0, 0)),
        pl.BlockSpec((128, 128), lambda i: (0, 0)),
        pl.BlockSpec((8, 128), lambda i: (0, 0)),
    ]
    return pl.pallas_call(
        body,
        grid=(nblk,),
        in_specs=specs,
        out_specs=pl.BlockSpec((BM, 128), lambda i: (i, 0)),
        out_shape=jax.ShapeDtypeStruct((M, 128), jnp.float32),
    )(pfs, fs, macc, W1p, W1f, W1g, b1_8, W2, b2_8, W3p, b3_8)


def _b8(b):
    return jnp.broadcast_to(b.reshape(1, -1), (8, b.shape[0]))


def kernel(x, spiral, Wp, W1a, b1a, W1b, b1b, Wd, Wr0a, br0a, Wr0b, br0b,
           Wr1a, br1a, Wr1b, br1b, Wr2a, br2a, Wr2b, br2b, Wo1, bo1, Wo2,
           bo2, Wo3, bo3):
    Bn, Vn, FIN = x.shape
    KP = 512
    xp = jnp.pad(x[0], ((0, VP - Vn), (0, KP - FIN)))
    Wpp = jnp.pad(Wp, ((0, KP - FIN), (0, 0)))
    idxf = jnp.pad(spiral.reshape(-1), (0, VP * L - spiral.size))
    idxf = idxf.astype(jnp.int32)

    gather256 = _make_sc_gather(256)
    gather128 = _make_sc_gather(128)

    pfs = _mm(xp, Wpp, relu=True)

    g = gather256(pfs, idxf).reshape(VP, L * 256)
    h = _mm(g, W1a, bias8=_b8(b1a), relu=True)
    g = gather256(h, idxf).reshape(VP, L * 256)
    fs = _mm(g, W1b, bias8=_b8(b1b), relu=True, res=pfs)

    fs, macc = _wd_and_max(fs, Wd)

    for (Wa, ba, Wb, bb) in ((Wr0a, br0a, Wr0b, br0b),
                             (Wr1a, br1a, Wr1b, br1b),
                             (Wr2a, br2a, Wr2b, br2b)):
        g = gather128(fs, idxf).reshape(VP, L * 128)
        h = _mm(g, Wa, bias8=_b8(ba), relu=True)
        g = gather128(h, idxf).reshape(VP, L * 128)
        fs = _mm(g, Wb, bias8=_b8(bb), relu=True, res=fs)

    W1p = Wo1[:256]
    W1f = Wo1[256:384]
    W1g = Wo1[384:]
    W3p = jnp.pad(Wo3, ((0, 0), (0, 128 - Wo3.shape[1])))
    b3p = jnp.pad(bo3, (0, 128 - bo3.shape[0]))
    o = _final_mlp(pfs, fs, macc, W1p, W1f, W1g, _b8(bo1), Wo2, _b8(bo2),
                   W3p, _b8(b3p))
    return o[:V, :3].reshape(1, V, 3)


# speedup vs baseline: 1.1319x; 1.0841x over previous
import functools

import jax
import jax.numpy as jnp
from jax import lax
from jax.experimental import pallas as pl
from jax.experimental.pallas import tpu as pltpu
from jax.experimental.pallas import tpu_sc as plsc

V = 10000
VP = 10240
L = 16
NW = 32
GK = 64
NB = 4


def _make_sc_gather(C: int):
    R = VP * L
    chunk = R // NW
    iters = chunk // GK
    rounds = iters // NB
    mesh = plsc.VectorSubcoreMesh(core_axis_name="c", subcore_axis_name="s")

    @functools.partial(
        pl.kernel,
        mesh=mesh,
        out_type=jax.ShapeDtypeStruct((R, C), jnp.float32),
        scratch_types=[
            pltpu.VMEM((chunk,), jnp.int32),
        ] + [pltpu.VMEM((GK, C), jnp.float32) for _ in range(NB)]
          + [pltpu.SemaphoreType.DMA for _ in range(2 * NB)],
    )
    def kfn(table_hbm, idx_hbm, g_hbm, idx_v, *rest):
        bufs = rest[:NB]
        gsems = rest[NB:2 * NB]
        wsems = rest[2 * NB:]
        wid = lax.axis_index("s") * 2 + lax.axis_index("c")
        base = pl.multiple_of(wid * chunk, GK)
        pltpu.sync_copy(idx_hbm.at[pl.ds(base, chunk)], idx_v)

        def start_gather(b, g):
            pltpu.async_copy(
                table_hbm.at[idx_v.at[pl.ds(g * GK, GK)]], bufs[b], gsems[b])

        def wait_gather(b):
            pltpu.make_async_copy(
                g_hbm.at[pl.ds(0, GK)], bufs[b], gsems[b]).wait()

        def start_wb(b, g):
            off = pl.multiple_of(base + g * GK, GK)
            pltpu.async_copy(bufs[b], g_hbm.at[pl.ds(off, GK)], wsems[b])

        def wait_wb(b):
            pltpu.make_async_copy(
                bufs[b], g_hbm.at[pl.ds(0, GK)], wsems[b]).wait()

        for b in range(NB):
            start_gather(b, b)

        def body(r, carry):
            for b in range(NB):
                g = r * NB + b
                wait_gather(b)
                start_wb(b, g)
            for b in range(NB):
                wait_wb(b)
                start_gather(b, (r + 1) * NB + b)
            return carry

        lax.fori_loop(0, rounds - 1, body, 0)
        for b in range(NB):
            g = (rounds - 1) * NB + b
            wait_gather(b)
            start_wb(b, g)
        for b in range(NB):
            wait_wb(b)

    return kfn


def _mm(xp, W, bias8=None, relu=False, res=None, mask=True, BM=256):
    M, K = xp.shape
    N = W.shape[1]
    nblk = M // BM
    args = [xp, W]
    in_specs = [
        pl.BlockSpec((BM, K), lambda i: (i, 0)),
        pl.BlockSpec((K, N), lambda i: (0, 0)),
    ]
    if bias8 is not None:
        args.append(bias8)
        in_specs.append(pl.BlockSpec((8, N), lambda i: (0, 0)))
    if res is not None:
        args.append(res)
        in_specs.append(pl.BlockSpec((BM, N), lambda i: (i, 0)))
    have_bias = bias8 is not None
    have_res = res is not None

    def body(*refs):
        x_ref, w_ref = refs[0], refs[1]
        rest = refs[2:-1]
        o_ref = refs[-1]
        y = jnp.dot(x_ref[...], w_ref[...], preferred_element_type=jnp.float32)
        ri = 0
        if have_bias:
            y = y + rest[0][0:1, :]
            ri = 1
        if mask:
            i = pl.program_id(0)
            rows = i * BM + lax.broadcasted_iota(jnp.int32, (BM, 1), 0)
            y = jnp.where(rows < V, y, 0.0)
        if have_res:
            y = y + rest[ri][...]
        if relu:
            y = jnp.maximum(y, 0.0)
        o_ref[...] = y

    return pl.pallas_call(
        body,
        grid=(nblk,),
        in_specs=in_specs,
        out_specs=pl.BlockSpec((BM, N), lambda i: (i, 0)),
        out_shape=jax.ShapeDtypeStruct((M, N), jnp.float32),
    )(*args)


def _wd_and_max(fs, Wd, BM=256):
    M, K = fs.shape
    N = Wd.shape[1]
    nblk = M // BM

    def body(x_ref, w_ref, o_ref, m_ref):
        i = pl.program_id(0)
        y = jnp.dot(x_ref[...], w_ref[...], preferred_element_type=jnp.float32)
        rows = i * BM + lax.broadcasted_iota(jnp.int32, (BM, 1), 0)
        valid = rows < V
        o_ref[...] = jnp.where(valid, y, 0.0)
        ym = jnp.where(valid, y, -1e30)
        m = ym[0:8]
        for j in range(1, BM // 8):
            m = jnp.maximum(m, ym[j * 8:(j + 1) * 8])

        @pl.when(i == 0)
        def _():
            m_ref[...] = m

        @pl.when(i > 0)
        def _():
            m_ref[...] = jnp.maximum(m_ref[...], m)

    return pl.pallas_call(
        body,
        grid=(nblk,),
        in_specs=[
            pl.BlockSpec((BM, K), lambda i: (i, 0)),
            pl.BlockSpec((K, N), lambda i: (0, 0)),
        ],
        out_specs=[
            pl.BlockSpec((BM, N), lambda i: (i, 0)),
            pl.BlockSpec((8, N), lambda i: (0, 0)),
        ],
        out_shape=[
            jax.ShapeDtypeStruct((M, N), jnp.float32),
            jax.ShapeDtypeStruct((8, N), jnp.float32),
        ],
    )(fs, Wd)


def _final_mlp(pfs, fs, macc, W1p, W1f, W1g, b1_8, W2, b2_8, W3p, b3_8, BM=256):
    M = pfs.shape[0]
    nblk = M // BM

    def body(p_ref, f_ref, g_ref, w1p, w1f, w1g, b1, w2, b2, w3, b3, o_ref):
        gmax = jnp.max(g_ref[...], axis=0, keepdims=True)
        gc = jnp.dot(gmax, w1g[...], preferred_element_type=jnp.float32)
        o1 = jnp.dot(p_ref[...], w1p[...], preferred_element_type=jnp.float32)
        o1 = o1 + jnp.dot(f_ref[...], w1f[...],
                          preferred_element_type=jnp.float32)
        o1 = jnp.maximum(o1 + gc + b1[0:1, :], 0.0)
        o2 = jnp.maximum(
            jnp.dot(o1, w2[...], preferred_element_type=jnp.float32)
            + b2[0:1, :], 0.0)
        o_ref[...] = (
            jnp.dot(o2, w3[...], preferred_element_type=jnp.float32)
            + b3[0:1, :])

    specs = [
        pl.BlockSpec((BM, 256), lambda i: (i, 0)),
        pl.BlockSpec((BM, 128), lambda i: (i, 0)),
        pl.BlockSpec((8, 128), lambda i: (0, 0)),
        pl.BlockSpec((256, 256), lambda i: (0, 0)),
        pl.BlockSpec((128, 256), lambda i: (0, 0)),
        pl.BlockSpec((128, 256), lambda i: (0, 0)),
        pl.BlockSpec((8, 256), lambda i: (0, 0)),
        pl.BlockSpec((256, 128), lambda i: (0, 0)),
        pl.BlockSpec((8, 128), lambda i: (0, 0)),
        pl.BlockSpec((128, 128), lambda i: (0, 0)),
        pl.BlockSpec((8, 128), lambda i: (0, 0)),
    ]
    return pl.pallas_call(
        body,
        grid=(nblk,),
        in_specs=specs,
        out_specs=pl.BlockSpec((BM, 128), lambda i: (i, 0)),
        out_shape=jax.ShapeDtypeStruct((M, 128), jnp.float32),
    )(pfs, fs, macc, W1p, W1f, W1g, b1_8, W2, b2_8, W3p, b3_8)


def _b8(b):
    return jnp.broadcast_to(b.reshape(1, -1), (8, b.shape[0]))


def kernel(x, spiral, Wp, W1a, b1a, W1b, b1b, Wd, Wr0a, br0a, Wr0b, br0b,
           Wr1a, br1a, Wr1b, br1b, Wr2a, br2a, Wr2b, br2b, Wo1, bo1, Wo2,
           bo2, Wo3, bo3):
    Bn, Vn, FIN = x.shape
    KP = 512
    xp = jnp.pad(x[0], ((0, VP - Vn), (0, KP - FIN)))
    Wpp = jnp.pad(Wp, ((0, KP - FIN), (0, 0)))
    idxf = jnp.pad(spiral.reshape(-1), (0, VP * L - spiral.size))
    idxf = idxf.astype(jnp.int32)

    gather256 = _make_sc_gather(256)
    gather128 = _make_sc_gather(128)

    pfs = _mm(xp, Wpp, relu=True)

    g = gather256(pfs, idxf).reshape(VP, L * 256)
    h = _mm(g, W1a, bias8=_b8(b1a), relu=True)
    g = gather256(h, idxf).reshape(VP, L * 256)
    fs = _mm(g, W1b, bias8=_b8(b1b), relu=True, res=pfs)

    fs, macc = _wd_and_max(fs, Wd)

    for (Wa, ba, Wb, bb) in ((Wr0a, br0a, Wr0b, br0b),
                             (Wr1a, br1a, Wr1b, br1b),
                             (Wr2a, br2a, Wr2b, br2b)):
        g = gather128(fs, idxf).reshape(VP, L * 128)
        h = _mm(g, Wa, bias8=_b8(ba), relu=True)
        g = gather128(h, idxf).reshape(VP, L * 128)
        fs = _mm(g, Wb, bias8=_b8(bb), relu=True, res=fs)

    W1p = Wo1[:256]
    W1f = Wo1[256:384]
    W1g = Wo1[384:]
    W3p = jnp.pad(Wo3, ((0, 0), (0, 128 - Wo3.shape[1])))
    b3p = jnp.pad(bo3, (0, 128 - bo3.shape[0]))
    o = _final_mlp(pfs, fs, macc, W1p, W1f, W1g, _b8(bo1), Wo2, _b8(bo2),
                   W3p, _b8(b3p))
    return o[:V, :3].reshape(1, V, 3)
